# baseline (device time: 148319 ns/iter reference)
import jax
import jax.numpy as jnp
from jax import lax
from jax.experimental import pallas as pl
from jax.experimental.pallas import tpu as pltpu

N_DEV = 16
B, SQ, HQ, DH = 2, 512, 8, 64
HD = HQ * DH
DM = 768
CHUNK = SQ // N_DEV


def _body(x_ref, wq_ref, k_ref, v_ref, wo_ref, out_ref,
          o_acc, s_acc, o_comm, s_comm, out_comm,
          o_send, o_recv, s_send, s_recv, ag_send, ag_recv):
    my = lax.axis_index("i")
    left = (my - 1) % N_DEV
    right = (my + 1) % N_DEV

    qi = lax.broadcasted_iota(jnp.int32, (SQ, SQ), 0)
    ki = lax.broadcasted_iota(jnp.int32, (SQ, SQ), 1)
    mask = ((qi // 64) % 4) == ((ki // 64) % 4)

    for b in range(B):
        qb = jnp.dot(x_ref[b], wq_ref[:, :], preferred_element_type=jnp.float32)
        for h in range(HQ):
            g = b * HQ + h
            qh = qb[:, h * DH:(h + 1) * DH]
            kh = k_ref[b][:, h * DH:(h + 1) * DH]
            vh = v_ref[b][:, h * DH:(h + 1) * DH]
            scores = lax.dot_general(
                qh, kh, (((1,), (1,)), ((), ())),
                preferred_element_type=jnp.float32) * 0.125
            w = jnp.where(mask, jnp.exp(scores), 0.0)
            o_acc[g] = jnp.dot(w, vh, preferred_element_type=jnp.float32)
            s_acc[:, g:g + 1] = jnp.sum(w, axis=1, keepdims=True)

    barrier = pltpu.get_barrier_semaphore()
    for nbr in (left, right):
        pl.semaphore_signal(barrier, inc=1, device_id=(nbr,),
                            device_id_type=pl.DeviceIdType.MESH)
    pl.semaphore_wait(barrier, 2)

    c0 = my % N_DEV
    o_comm[0] = o_acc[:, pl.ds(c0 * CHUNK, CHUNK), :]
    s_comm[0] = s_acc[pl.ds(c0 * CHUNK, CHUNK), :]
    for t in range(N_DEV - 1):
        s_slot = t % 2
        r_slot = (t + 1) % 2
        rd_o = pltpu.make_async_remote_copy(
            src_ref=o_comm.at[s_slot], dst_ref=o_comm.at[r_slot],
            send_sem=o_send.at[s_slot], recv_sem=o_recv.at[r_slot],
            device_id=(right,), device_id_type=pl.DeviceIdType.MESH)
        rd_s = pltpu.make_async_remote_copy(
            src_ref=s_comm.at[s_slot], dst_ref=s_comm.at[r_slot],
            send_sem=s_send.at[s_slot], recv_sem=s_recv.at[r_slot],
            device_id=(right,), device_id_type=pl.DeviceIdType.MESH)
        rd_o.start()
        rd_s.start()
        rd_o.wait()
        rd_s.wait()
        c_recv = (my - t - 1) % N_DEV
        o_comm[r_slot] = o_comm[r_slot] + o_acc[:, pl.ds(c_recv * CHUNK, CHUNK), :]
        s_comm[r_slot] = s_comm[r_slot] + s_acc[pl.ds(c_recv * CHUNK, CHUNK), :]

    fin = (N_DEV - 1) % 2
    qc = (my + 1) % N_DEV
    of = o_comm[fin]
    sf = s_comm[fin]
    for b in range(B):
        acc = jnp.zeros((CHUNK, DM), jnp.float32)
        for h in range(HQ):
            g = b * HQ + h
            ctx = of[g] / sf[:, g:g + 1]
            acc = acc + jnp.dot(ctx, wo_ref[h * DH:(h + 1) * DH, :],
                                preferred_element_type=jnp.float32)
        out_ref[b, pl.ds(qc * CHUNK, CHUNK), :] = acc
        out_comm[0, b] = acc

    for h in range(N_DEV - 1):
        s_slot = h % 2
        r_slot = (h + 1) % 2
        rd = pltpu.make_async_remote_copy(
            src_ref=out_comm.at[s_slot], dst_ref=out_comm.at[r_slot],
            send_sem=ag_send.at[s_slot], recv_sem=ag_recv.at[r_slot],
            device_id=(right,), device_id_type=pl.DeviceIdType.MESH)
        rd.start()
        rd.wait()
        origin = (my - h) % N_DEV
        out_ref[:, pl.ds(origin * CHUNK, CHUNK), :] = out_comm[r_slot]


def kernel(x, Wq, K_ext, V_ext, Wo):
    k2 = K_ext.reshape(B, SQ, HD)
    v2 = V_ext.reshape(B, SQ, HD)
    return pl.pallas_call(
        _body,
        out_shape=jax.ShapeDtypeStruct((B, SQ, DM), jnp.float32),
        in_specs=[pl.BlockSpec(memory_space=pltpu.VMEM)] * 5,
        out_specs=pl.BlockSpec(memory_space=pltpu.VMEM),
        scratch_shapes=[
            pltpu.VMEM((B * HQ, SQ, DH), jnp.float32),
            pltpu.VMEM((SQ, B * HQ), jnp.float32),
            pltpu.VMEM((2, B * HQ, CHUNK, DH), jnp.float32),
            pltpu.VMEM((2, CHUNK, B * HQ), jnp.float32),
            pltpu.VMEM((2, B, CHUNK, DM), jnp.float32),
            pltpu.SemaphoreType.DMA((2,)),
            pltpu.SemaphoreType.DMA((2,)),
            pltpu.SemaphoreType.DMA((2,)),
            pltpu.SemaphoreType.DMA((2,)),
            pltpu.SemaphoreType.DMA((2,)),
            pltpu.SemaphoreType.DMA((2,)),
        ],
        compiler_params=pltpu.CompilerParams(collective_id=0),
    )(x, Wq, k2, v2, Wo)


# device time: 124742 ns/iter; 1.1890x vs baseline; 1.1890x over previous
import jax
import jax.numpy as jnp
from jax import lax
from jax.experimental import pallas as pl
from jax.experimental.pallas import tpu as pltpu

N_DEV = 16
B, SQ, HQ, DH = 2, 512, 8, 64
G = B * HQ
HD = HQ * DH
DM = 768
CHUNK = SQ // N_DEV


def _body(x_ref, wq_ref, k_ref, v_ref, wo_ref, out_ref,
          o_acc, s_acc, o_stage, s_stage, o_rbuf, s_rbuf,
          ctx_stage, ctx_rbuf, ctx_full,
          o_ssem, o_rsem, s_ssem, s_rsem, ag_ssem, ag_rsem):
    my = lax.axis_index("i")

    qi = lax.broadcasted_iota(jnp.int32, (SQ, SQ), 0)
    ki = lax.broadcasted_iota(jnp.int32, (SQ, SQ), 1)
    mask = ((qi // 64) % 4) == ((ki // 64) % 4)

    for b in range(B):
        qb = jnp.dot(x_ref[b], wq_ref[:, :], preferred_element_type=jnp.float32)
        for h in range(HQ):
            g = b * HQ + h
            qh = qb[:, h * DH:(h + 1) * DH]
            kh = k_ref[b][:, h * DH:(h + 1) * DH]
            vh = v_ref[b][:, h * DH:(h + 1) * DH]
            scores = lax.dot_general(
                qh, kh, (((1,), (1,)), ((), ())),
                preferred_element_type=jnp.float32) * 0.125
            w = jnp.where(mask, jnp.exp(scores), 0.0)
            o_acc[g] = jnp.dot(w, vh, preferred_element_type=jnp.float32)
            s_acc[:, g:g + 1] = jnp.sum(w, axis=1, keepdims=True)

    for e in range(1, N_DEV):
        tgt = (my + e) % N_DEV
        o_stage[e] = o_acc[:, pl.ds(tgt * CHUNK, CHUNK), :]
        s_stage[e] = s_acc[pl.ds(tgt * CHUNK, CHUNK), :]

    barrier = pltpu.get_barrier_semaphore()
    for e in range(1, N_DEV):
        pl.semaphore_signal(barrier, inc=1, device_id=((my + e) % N_DEV,),
                            device_id_type=pl.DeviceIdType.MESH)
    pl.semaphore_wait(barrier, N_DEV - 1)

    rs_rdmas = []
    for e in range(1, N_DEV):
        tgt = (my + e) % N_DEV
        rd_o = pltpu.make_async_remote_copy(
            src_ref=o_stage.at[e], dst_ref=o_rbuf.at[e],
            send_sem=o_ssem.at[e], recv_sem=o_rsem.at[e],
            device_id=(tgt,), device_id_type=pl.DeviceIdType.MESH)
        rd_s = pltpu.make_async_remote_copy(
            src_ref=s_stage.at[e], dst_ref=s_rbuf.at[e],
            send_sem=s_ssem.at[e], recv_sem=s_rsem.at[e],
            device_id=(tgt,), device_id_type=pl.DeviceIdType.MESH)
        rd_o.start()
        rd_s.start()
        rs_rdmas.append((rd_o, rd_s))

    ctx_stage[...] = o_acc[:, pl.ds(my * CHUNK, CHUNK), :]
    s_sum = s_acc[pl.ds(my * CHUNK, CHUNK), :]
    for e in range(1, N_DEV):
        rd_o, rd_s = rs_rdmas[e - 1]
        rd_o.wait_recv()
        rd_s.wait_recv()
        ctx_stage[...] = ctx_stage[...] + o_rbuf[e]
        s_sum = s_sum + s_rbuf[e]

    for g in range(G):
        ctx_stage[g] = ctx_stage[g] / s_sum[:, g:g + 1]

    ag_rdmas = []
    for e in range(1, N_DEV):
        tgt = (my + e) % N_DEV
        rd = pltpu.make_async_remote_copy(
            src_ref=ctx_stage, dst_ref=ctx_rbuf.at[e],
            send_sem=ag_ssem.at[e], recv_sem=ag_rsem.at[e],
            device_id=(tgt,), device_id_type=pl.DeviceIdType.MESH)
        rd.start()
        ag_rdmas.append(rd)

    ctx_full[:, pl.ds(my * CHUNK, CHUNK), :] = ctx_stage[...]
    for e in range(1, N_DEV):
        ag_rdmas[e - 1].wait_recv()
        src = (my - e) % N_DEV
        ctx_full[:, pl.ds(src * CHUNK, CHUNK), :] = ctx_rbuf[e]

    for b in range(B):
        acc = jnp.zeros((SQ, DM), jnp.float32)
        for h in range(HQ):
            g = b * HQ + h
            acc = acc + jnp.dot(ctx_full[g], wo_ref[h * DH:(h + 1) * DH, :],
                                preferred_element_type=jnp.float32)
        out_ref[b] = acc

    for rd_o, rd_s in rs_rdmas:
        rd_o.wait_send()
        rd_s.wait_send()
    for rd in ag_rdmas:
        rd.wait_send()


def kernel(x, Wq, K_ext, V_ext, Wo):
    k2 = K_ext.reshape(B, SQ, HD)
    v2 = V_ext.reshape(B, SQ, HD)
    return pl.pallas_call(
        _body,
        out_shape=jax.ShapeDtypeStruct((B, SQ, DM), jnp.float32),
        in_specs=[pl.BlockSpec(memory_space=pltpu.VMEM)] * 5,
        out_specs=pl.BlockSpec(memory_space=pltpu.VMEM),
        scratch_shapes=[
            pltpu.VMEM((G, SQ, DH), jnp.float32),
            pltpu.VMEM((SQ, G), jnp.float32),
            pltpu.VMEM((N_DEV, G, CHUNK, DH), jnp.float32),
            pltpu.VMEM((N_DEV, CHUNK, G), jnp.float32),
            pltpu.VMEM((N_DEV, G, CHUNK, DH), jnp.float32),
            pltpu.VMEM((N_DEV, CHUNK, G), jnp.float32),
            pltpu.VMEM((G, CHUNK, DH), jnp.float32),
            pltpu.VMEM((N_DEV, G, CHUNK, DH), jnp.float32),
            pltpu.VMEM((G, SQ, DH), jnp.float32),
            pltpu.SemaphoreType.DMA((N_DEV,)),
            pltpu.SemaphoreType.DMA((N_DEV,)),
            pltpu.SemaphoreType.DMA((N_DEV,)),
            pltpu.SemaphoreType.DMA((N_DEV,)),
            pltpu.SemaphoreType.DMA((N_DEV,)),
            pltpu.SemaphoreType.DMA((N_DEV,)),
        ],
        compiler_params=pltpu.CompilerParams(collective_id=0),
    )(x, Wq, k2, v2, Wo)


# device time: 70545 ns/iter; 2.1025x vs baseline; 1.7683x over previous
import jax
import jax.numpy as jnp
from jax import lax
from jax.experimental import pallas as pl
from jax.experimental.pallas import tpu as pltpu

N_DEV = 16
COMM = True
B, SQ, HQ, DH = 2, 512, 8, 64
G = B * HQ
HD = HQ * DH
DM = 768
CHUNK = SQ // N_DEV
BF16 = jnp.bfloat16


def _body(x_ref, wq_ref, k_ref, v_ref, wo_ref, out_ref,
          o_acc, s_acc, o_stage, s_stage, o_rbuf, s_rbuf,
          ctx_bcast, ctx_rbuf, ctx_full,
          o_ssem, o_rsem, s_ssem, s_rsem, ag_ssem, ag_rsem):
    my = lax.axis_index("i")

    qi = lax.broadcasted_iota(jnp.int32, (SQ, SQ), 0)
    ki = lax.broadcasted_iota(jnp.int32, (SQ, SQ), 1)
    mask = ((qi // 64) % 4) == ((ki // 64) % 4)

    for b in range(B):
        qb = jnp.dot(x_ref[b], wq_ref[:, :], preferred_element_type=jnp.float32)
        for h in range(HQ):
            g = b * HQ + h
            qh = qb[:, h * DH:(h + 1) * DH]
            kh = k_ref[b][:, h * DH:(h + 1) * DH]
            vh = v_ref[b][:, h * DH:(h + 1) * DH]
            scores = lax.dot_general(
                qh, kh, (((1,), (1,)), ((), ())),
                preferred_element_type=jnp.float32) * 0.125
            w = jnp.where(mask, jnp.exp(scores), 0.0)
            o_acc[g] = jnp.dot(w, vh, preferred_element_type=jnp.float32)
            s_acc[:, g:g + 1] = jnp.sum(w, axis=1, keepdims=True)

    for e in range(1, N_DEV):
        tgt = (my + e) % N_DEV
        o_stage[e] = o_acc[:, pl.ds(tgt * CHUNK, CHUNK), :].astype(BF16)
        s_stage[e] = s_acc[pl.ds(tgt * CHUNK, CHUNK), :]

    if COMM:
        barrier = pltpu.get_barrier_semaphore()
        for e in range(1, N_DEV):
            pl.semaphore_signal(barrier, inc=1, device_id=((my + e) % N_DEV,),
                                device_id_type=pl.DeviceIdType.MESH)
        pl.semaphore_wait(barrier, N_DEV - 1)

    rs_rdmas = []
    if COMM:
        for e in range(1, N_DEV):
            tgt = (my + e) % N_DEV
            rd_o = pltpu.make_async_remote_copy(
                src_ref=o_stage.at[e], dst_ref=o_rbuf.at[e],
                send_sem=o_ssem.at[e], recv_sem=o_rsem.at[e],
                device_id=(tgt,), device_id_type=pl.DeviceIdType.MESH)
            rd_s = pltpu.make_async_remote_copy(
                src_ref=s_stage.at[e], dst_ref=s_rbuf.at[e],
                send_sem=s_ssem.at[e], recv_sem=s_rsem.at[e],
                device_id=(tgt,), device_id_type=pl.DeviceIdType.MESH)
            rd_o.start()
            rd_s.start()
            rs_rdmas.append((rd_o, rd_s))

    o_sum = o_acc[:, pl.ds(my * CHUNK, CHUNK), :]
    s_sum = s_acc[pl.ds(my * CHUNK, CHUNK), :]
    for e in range(1, N_DEV):
        if COMM:
            rd_o, rd_s = rs_rdmas[e - 1]
            rd_o.wait_recv()
            rd_s.wait_recv()
        o_sum = o_sum + o_rbuf[e].astype(jnp.float32)
        s_sum = s_sum + s_rbuf[e]

    for g in range(G):
        ctx_bcast[g] = (o_sum[g] / s_sum[:, g:g + 1]).astype(BF16)

    ag_rdmas = []
    if COMM:
        for e in range(1, N_DEV):
            tgt = (my + e) % N_DEV
            rd = pltpu.make_async_remote_copy(
                src_ref=ctx_bcast, dst_ref=ctx_rbuf.at[e],
                send_sem=ag_ssem.at[e], recv_sem=ag_rsem.at[e],
                device_id=(tgt,), device_id_type=pl.DeviceIdType.MESH)
            rd.start()
            ag_rdmas.append(rd)

    ctx_full[:, pl.ds(my * CHUNK, CHUNK), :] = ctx_bcast[...]
    for e in range(1, N_DEV):
        if COMM:
            ag_rdmas[e - 1].wait_recv()
        src = (my - e) % N_DEV
        ctx_full[:, pl.ds(src * CHUNK, CHUNK), :] = ctx_rbuf[e]

    wo_bf = wo_ref[:, :].astype(BF16)
    for b in range(B):
        acc = jnp.zeros((SQ, DM), jnp.float32)
        for h in range(HQ):
            g = b * HQ + h
            acc = acc + jnp.dot(ctx_full[g], wo_bf[h * DH:(h + 1) * DH, :],
                                preferred_element_type=jnp.float32)
        out_ref[b] = acc

    if COMM:
        for rd_o, rd_s in rs_rdmas:
            rd_o.wait_send()
            rd_s.wait_send()
        for rd in ag_rdmas:
            rd.wait_send()


def kernel(x, Wq, K_ext, V_ext, Wo):
    k2 = K_ext.reshape(B, SQ, HD)
    v2 = V_ext.reshape(B, SQ, HD)
    return pl.pallas_call(
        _body,
        out_shape=jax.ShapeDtypeStruct((B, SQ, DM), jnp.float32),
        in_specs=[pl.BlockSpec(memory_space=pltpu.VMEM)] * 5,
        out_specs=pl.BlockSpec(memory_space=pltpu.VMEM),
        scratch_shapes=[
            pltpu.VMEM((G, SQ, DH), jnp.float32),
            pltpu.VMEM((SQ, G), jnp.float32),
            pltpu.VMEM((N_DEV, G, CHUNK, DH), BF16),
            pltpu.VMEM((N_DEV, CHUNK, G), jnp.float32),
            pltpu.VMEM((N_DEV, G, CHUNK, DH), BF16),
            pltpu.VMEM((N_DEV, CHUNK, G), jnp.float32),
            pltpu.VMEM((G, CHUNK, DH), BF16),
            pltpu.VMEM((N_DEV, G, CHUNK, DH), BF16),
            pltpu.VMEM((G, SQ, DH), BF16),
            pltpu.SemaphoreType.DMA((N_DEV,)),
            pltpu.SemaphoreType.DMA((N_DEV,)),
            pltpu.SemaphoreType.DMA((N_DEV,)),
            pltpu.SemaphoreType.DMA((N_DEV,)),
            pltpu.SemaphoreType.DMA((N_DEV,)),
            pltpu.SemaphoreType.DMA((N_DEV,)),
        ],
        compiler_params=(pltpu.CompilerParams(collective_id=0)
                         if COMM else pltpu.CompilerParams()),
    )(x, Wq, k2, v2, Wo)


# device time: 67006 ns/iter; 2.2135x vs baseline; 1.0528x over previous
import jax
import jax.numpy as jnp
from jax import lax
from jax.experimental import pallas as pl
from jax.experimental.pallas import tpu as pltpu

N_DEV = 16
COMM = True
B, SQ, HQ, DH = 2, 512, 8, 64
G = B * HQ
HD = HQ * DH
DM = 768
CHUNK = SQ // N_DEV
BF16 = jnp.bfloat16


def _body(x_ref, wq_ref, k_ref, v_ref, wo_ref, out_ref,
          o_acc, s_acc, o_stage, o_rbuf,
          ctx_bcast, ctx_rbuf, ctx_full,
          o_ssem, o_rsem, ag_ssem, ag_rsem):
    my = lax.axis_index("i")

    if COMM:
        barrier = pltpu.get_barrier_semaphore()
        for e in range(1, N_DEV):
            pl.semaphore_signal(barrier, inc=1, device_id=((my + e) % N_DEV,),
                                device_id_type=pl.DeviceIdType.MESH)

    qi = lax.broadcasted_iota(jnp.int32, (SQ, SQ), 0)
    ki = lax.broadcasted_iota(jnp.int32, (SQ, SQ), 1)
    mask = ((qi // 64) % 4) == ((ki // 64) % 4)

    wq_bf = wq_ref[:, :].astype(BF16)
    for b in range(B):
        qp = jnp.dot(x_ref[b].astype(BF16), wq_bf,
                     preferred_element_type=jnp.float32)
        qb_bf = qp.astype(BF16)
        kb_bf = k_ref[b].astype(BF16)
        vb_bf = v_ref[b].astype(BF16)
        for h in range(HQ):
            g = b * HQ + h
            qh = qb_bf[:, h * DH:(h + 1) * DH]
            kh = kb_bf[:, h * DH:(h + 1) * DH]
            vh = vb_bf[:, h * DH:(h + 1) * DH]
            scores = lax.dot_general(
                qh, kh, (((1,), (1,)), ((), ())),
                preferred_element_type=jnp.float32) * 0.125
            w = jnp.where(mask, jnp.exp(scores), 0.0)
            o_acc[g] = jnp.dot(w.astype(BF16), vh,
                               preferred_element_type=jnp.float32)
            s_acc[:, g:g + 1] = jnp.sum(w, axis=1, keepdims=True)

    for e in range(1, N_DEV):
        tgt = (my + e) % N_DEV
        o_stage[e, 0:G] = o_acc[:, pl.ds(tgt * CHUNK, CHUNK), :].astype(BF16)
        o_stage[e, G, :, 0:G] = s_acc[pl.ds(tgt * CHUNK, CHUNK), :].astype(BF16)

    if COMM:
        pl.semaphore_wait(barrier, N_DEV - 1)

    rs_rdmas = []
    if COMM:
        for e in range(1, N_DEV):
            tgt = (my + e) % N_DEV
            rd = pltpu.make_async_remote_copy(
                src_ref=o_stage.at[e], dst_ref=o_rbuf.at[e],
                send_sem=o_ssem.at[e], recv_sem=o_rsem.at[e],
                device_id=(tgt,), device_id_type=pl.DeviceIdType.MESH)
            rd.start()
            rs_rdmas.append(rd)

    o_sum = o_acc[:, pl.ds(my * CHUNK, CHUNK), :]
    s_sum = s_acc[pl.ds(my * CHUNK, CHUNK), :]
    for e in range(1, N_DEV):
        if COMM:
            rs_rdmas[e - 1].wait_recv()
        o_sum = o_sum + o_rbuf[e, 0:G].astype(jnp.float32)
        s_sum = s_sum + o_rbuf[e, G, :, 0:G].astype(jnp.float32)

    for g in range(G):
        ctx_bcast[g] = (o_sum[g] / s_sum[:, g:g + 1]).astype(BF16)

    ag_rdmas = []
    if COMM:
        for e in range(1, N_DEV):
            tgt = (my + e) % N_DEV
            rd = pltpu.make_async_remote_copy(
                src_ref=ctx_bcast, dst_ref=ctx_rbuf.at[e],
                send_sem=ag_ssem.at[e], recv_sem=ag_rsem.at[e],
                device_id=(tgt,), device_id_type=pl.DeviceIdType.MESH)
            rd.start()
            ag_rdmas.append(rd)

    ctx_full[:, pl.ds(my * CHUNK, CHUNK), :] = ctx_bcast[...]
    for e in range(1, N_DEV):
        if COMM:
            ag_rdmas[e - 1].wait_recv()
        src = (my - e) % N_DEV
        ctx_full[:, pl.ds(src * CHUNK, CHUNK), :] = ctx_rbuf[e]

    wo_bf = wo_ref[:, :].astype(BF16)
    for b in range(B):
        acc = jnp.zeros((SQ, DM), jnp.float32)
        for h in range(HQ):
            g = b * HQ + h
            acc = acc + jnp.dot(ctx_full[g], wo_bf[h * DH:(h + 1) * DH, :],
                                preferred_element_type=jnp.float32)
        out_ref[b] = acc

    if COMM:
        for rd in rs_rdmas:
            rd.wait_send()
        for rd in ag_rdmas:
            rd.wait_send()


def kernel(x, Wq, K_ext, V_ext, Wo):
    k2 = K_ext.reshape(B, SQ, HD)
    v2 = V_ext.reshape(B, SQ, HD)
    return pl.pallas_call(
        _body,
        out_shape=jax.ShapeDtypeStruct((B, SQ, DM), jnp.float32),
        in_specs=[pl.BlockSpec(memory_space=pltpu.VMEM)] * 5,
        out_specs=pl.BlockSpec(memory_space=pltpu.VMEM),
        scratch_shapes=[
            pltpu.VMEM((G, SQ, DH), jnp.float32),
            pltpu.VMEM((SQ, G), jnp.float32),
            pltpu.VMEM((N_DEV, G + 1, CHUNK, DH), BF16),
            pltpu.VMEM((N_DEV, G + 1, CHUNK, DH), BF16),
            pltpu.VMEM((G, CHUNK, DH), BF16),
            pltpu.VMEM((N_DEV, G, CHUNK, DH), BF16),
            pltpu.VMEM((G, SQ, DH), BF16),
            pltpu.SemaphoreType.DMA((N_DEV,)),
            pltpu.SemaphoreType.DMA((N_DEV,)),
            pltpu.SemaphoreType.DMA((N_DEV,)),
            pltpu.SemaphoreType.DMA((N_DEV,)),
        ],
        compiler_params=(pltpu.CompilerParams(collective_id=0)
                         if COMM else pltpu.CompilerParams()),
    )(x, Wq, k2, v2, Wo)


# device time: 62032 ns/iter; 2.3910x vs baseline; 1.0802x over previous
import jax
import jax.numpy as jnp
from jax import lax
from jax.experimental import pallas as pl
from jax.experimental.pallas import tpu as pltpu

N_DEV = 16
NP = 4
NJ = 4
B, SQ, HQ, DH = 2, 512, 8, 64
G = B * HQ
G1 = G + 1
HD = HQ * DH
DM = 768
CHUNK = SQ // N_DEV
BF16 = jnp.bfloat16
F32 = jnp.float32


def _body(x_ref, wq_ref, k_ref, v_ref, wo_ref, out_ref,
          o_acc, s_acc, a_stage, a_rbuf, gacc, b_stage, b_rbuf,
          ctx_bcast, cb_rbuf, cgrp, ca_rbuf, ctx_full,
          a_ssem, a_rsem, b_ssem, b_rsem,
          cb_ssem, cb_rsem, ca_ssem, ca_rsem):
    my = lax.axis_index("i")
    p = my // NJ
    j = my % NJ

    barrier = pltpu.get_barrier_semaphore()
    for m in range(1, NJ):
        pl.semaphore_signal(barrier, inc=1,
                            device_id=(p * NJ + (j + m) % NJ,),
                            device_id_type=pl.DeviceIdType.MESH)
        pl.semaphore_signal(barrier, inc=1,
                            device_id=(((p + m) % NP) * NJ + j,),
                            device_id_type=pl.DeviceIdType.MESH)

    qi = lax.broadcasted_iota(jnp.int32, (SQ, SQ), 0)
    ki = lax.broadcasted_iota(jnp.int32, (SQ, SQ), 1)
    mask = ((qi // 64) % 4) == ((ki // 64) % 4)

    wq_bf = wq_ref[:, :].astype(BF16)
    for b in range(B):
        qp_ = jnp.dot(x_ref[b].astype(BF16), wq_bf,
                      preferred_element_type=F32)
        qb_bf = qp_.astype(BF16)
        kb_bf = k_ref[b].astype(BF16)
        vb_bf = v_ref[b].astype(BF16)
        for h in range(HQ):
            g = b * HQ + h
            qh = qb_bf[:, h * DH:(h + 1) * DH]
            kh = kb_bf[:, h * DH:(h + 1) * DH]
            vh = vb_bf[:, h * DH:(h + 1) * DH]
            scores = lax.dot_general(
                qh, kh, (((1,), (1,)), ((), ())),
                preferred_element_type=F32) * 0.125
            w = jnp.where(mask, jnp.exp(scores), 0.0)
            o_acc[g] = jnp.dot(w.astype(BF16), vh,
                               preferred_element_type=F32)
            s_acc[:, g:g + 1] = jnp.sum(w, axis=1, keepdims=True)

    for m in range(1, NJ):
        jt = (j + m) % NJ
        for q in range(NP):
            c = NP * q + jt
            a_stage[m, q * G1:q * G1 + G] = (
                o_acc[:, pl.ds(c * CHUNK, CHUNK), :].astype(BF16))
            a_stage[m, q * G1 + G, :, 0:G] = (
                s_acc[pl.ds(c * CHUNK, CHUNK), :].astype(BF16))

    pl.semaphore_wait(barrier, 6)

    a_rdmas = []
    for m in range(1, NJ):
        tgt = p * NJ + (j + m) % NJ
        rd = pltpu.make_async_remote_copy(
            src_ref=a_stage.at[m], dst_ref=a_rbuf.at[m],
            send_sem=a_ssem.at[m], recv_sem=a_rsem.at[m],
            device_id=(tgt,), device_id_type=pl.DeviceIdType.MESH)
        rd.start()
        a_rdmas.append(rd)

    for q in range(NP):
        c = NP * q + j
        gacc[q * G1:q * G1 + G] = o_acc[:, pl.ds(c * CHUNK, CHUNK), :]
        gacc[q * G1 + G, :, 0:G] = s_acc[pl.ds(c * CHUNK, CHUNK), :]
    for m in range(1, NJ):
        a_rdmas[m - 1].wait_recv()
        gacc[...] = gacc[...] + a_rbuf[m].astype(F32)

    for n in range(1, NP):
        pt = (p + n) % NP
        b_stage[n] = gacc[pl.ds(pt * G1, G1)].astype(BF16)
    b_rdmas = []
    for n in range(1, NP):
        tgt = ((p + n) % NP) * NJ + j
        rd = pltpu.make_async_remote_copy(
            src_ref=b_stage.at[n], dst_ref=b_rbuf.at[n],
            send_sem=b_ssem.at[n], recv_sem=b_rsem.at[n],
            device_id=(tgt,), device_id_type=pl.DeviceIdType.MESH)
        rd.start()
        b_rdmas.append(rd)

    cs = gacc[pl.ds(p * G1, G1)]
    for n in range(1, NP):
        b_rdmas[n - 1].wait_recv()
        cs = cs + b_rbuf[n].astype(F32)

    for g in range(G):
        ctx_bcast[g] = (cs[g] / cs[G, :, g:g + 1]).astype(BF16)

    cb_rdmas = []
    for n in range(1, NP):
        tgt = ((p + n) % NP) * NJ + j
        rd = pltpu.make_async_remote_copy(
            src_ref=ctx_bcast, dst_ref=cb_rbuf.at[n],
            send_sem=cb_ssem.at[n], recv_sem=cb_rsem.at[n],
            device_id=(tgt,), device_id_type=pl.DeviceIdType.MESH)
        rd.start()
        cb_rdmas.append(rd)

    cgrp[pl.ds(p * G, G)] = ctx_bcast[...]
    ctx_full[:, pl.ds(my * CHUNK, CHUNK), :] = ctx_bcast[...]
    for n in range(1, NP):
        cb_rdmas[n - 1].wait_recv()
        ps = (p - n) % NP
        cgrp[pl.ds(ps * G, G)] = cb_rbuf[n]
        ctx_full[:, pl.ds((NP * ps + j) * CHUNK, CHUNK), :] = cb_rbuf[n]

    ca_rdmas = []
    for m in range(1, NJ):
        tgt = p * NJ + (j + m) % NJ
        rd = pltpu.make_async_remote_copy(
            src_ref=cgrp, dst_ref=ca_rbuf.at[m],
            send_sem=ca_ssem.at[m], recv_sem=ca_rsem.at[m],
            device_id=(tgt,), device_id_type=pl.DeviceIdType.MESH)
        rd.start()
        ca_rdmas.append(rd)

    for m in range(1, NJ):
        ca_rdmas[m - 1].wait_recv()
        js = (j - m) % NJ
        for q in range(NP):
            ctx_full[:, pl.ds((NP * q + js) * CHUNK, CHUNK), :] = (
                ca_rbuf[m, q * G:(q + 1) * G])

    wo_bf = wo_ref[:, :].astype(BF16)
    for b in range(B):
        acc = jnp.zeros((SQ, DM), F32)
        for h in range(HQ):
            g = b * HQ + h
            acc = acc + jnp.dot(ctx_full[g], wo_bf[h * DH:(h + 1) * DH, :],
                                preferred_element_type=F32)
        out_ref[b] = acc

    for rd in a_rdmas + b_rdmas + cb_rdmas + ca_rdmas:
        rd.wait_send()


def kernel(x, Wq, K_ext, V_ext, Wo):
    k2 = K_ext.reshape(B, SQ, HD)
    v2 = V_ext.reshape(B, SQ, HD)
    return pl.pallas_call(
        _body,
        out_shape=jax.ShapeDtypeStruct((B, SQ, DM), jnp.float32),
        in_specs=[pl.BlockSpec(memory_space=pltpu.VMEM)] * 5,
        out_specs=pl.BlockSpec(memory_space=pltpu.VMEM),
        scratch_shapes=[
            pltpu.VMEM((G, SQ, DH), F32),
            pltpu.VMEM((SQ, G), F32),
            pltpu.VMEM((NJ, NP * G1, CHUNK, DH), BF16),
            pltpu.VMEM((NJ, NP * G1, CHUNK, DH), BF16),
            pltpu.VMEM((NP * G1, CHUNK, DH), F32),
            pltpu.VMEM((NP, G1, CHUNK, DH), BF16),
            pltpu.VMEM((NP, G1, CHUNK, DH), BF16),
            pltpu.VMEM((G, CHUNK, DH), BF16),
            pltpu.VMEM((NP, G, CHUNK, DH), BF16),
            pltpu.VMEM((NP * G, CHUNK, DH), BF16),
            pltpu.VMEM((NJ, NP * G, CHUNK, DH), BF16),
            pltpu.VMEM((G, SQ, DH), BF16),
            pltpu.SemaphoreType.DMA((NJ,)),
            pltpu.SemaphoreType.DMA((NJ,)),
            pltpu.SemaphoreType.DMA((NP,)),
            pltpu.SemaphoreType.DMA((NP,)),
            pltpu.SemaphoreType.DMA((NP,)),
            pltpu.SemaphoreType.DMA((NP,)),
            pltpu.SemaphoreType.DMA((NJ,)),
            pltpu.SemaphoreType.DMA((NJ,)),
        ],
        compiler_params=pltpu.CompilerParams(collective_id=0),
    )(x, Wq, k2, v2, Wo)
